# baseline (speedup 1.0000x reference)
import functools
import math

import jax
import jax.numpy as jnp
from jax import lax
from jax.experimental import pallas as pl
from jax.experimental.pallas import tpu as pltpu
from jax.experimental.pallas import tpu_sc as plsc

N = 10000
E = 160000
F_IN = 128
HID = 32
G = 64
LOG2 = math.log(2.0)

NW = 32
CHUNK = 128
C_PER_W = 40
E_PAD = NW * CHUNK * C_PER_W
N_SP = 10240
ROWS_PER_TILE = N_SP // 16

_f32 = jnp.float32


@functools.lru_cache(maxsize=None)
def _sc_gather_kernel(W):
    mesh = plsc.VectorSubcoreMesh(core_axis_name="c", subcore_axis_name="s")

    @functools.partial(
        pl.kernel, mesh=mesh,
        out_type=jax.ShapeDtypeStruct((E_PAD, W), _f32),
        compiler_params=pltpu.CompilerParams(use_tc_tiling_on_sc=False),
        scratch_types=[
            pltpu.VMEM((C_PER_W, CHUNK), jnp.int32),
            pltpu.VMEM((CHUNK, W), _f32),
            pltpu.SemaphoreType.DMA,
        ],
    )
    def k(table_hbm, idx_hbm, out_hbm, idxs_v, rows_v, sem):
        c = lax.axis_index("c")
        s = lax.axis_index("s")
        wid = s * 2 + c
        pltpu.sync_copy(idx_hbm.at[pl.ds(wid * C_PER_W, C_PER_W)], idxs_v)

        def body(j, carry):
            pltpu.async_copy(table_hbm.at[idxs_v.at[j]], rows_v, sem).wait()
            pltpu.sync_copy(rows_v, out_hbm.at[pl.ds((wid * C_PER_W + j) * CHUNK, CHUNK)])
            return carry

        lax.fori_loop(0, C_PER_W, body, 0)

    return k


def _sc_gather(table, idx2):
    return _sc_gather_kernel(table.shape[1])(table, idx2)


@functools.lru_cache(maxsize=None)
def _sc_scatter_kernel(W):
    mesh = plsc.VectorSubcoreMesh(core_axis_name="c", subcore_axis_name="s")

    @functools.partial(
        pl.kernel, mesh=mesh,
        out_type=jax.ShapeDtypeStruct((2, N_SP, W), _f32),
        compiler_params=pltpu.CompilerParams(use_tc_tiling_on_sc=False),
        scratch_types=[
            pltpu.VMEM((C_PER_W, CHUNK), jnp.int32),
            pltpu.VMEM((CHUNK, W), _f32),
            pltpu.VMEM_SHARED((N_SP, W), _f32),
        ],
    )
    def k(rows_hbm, idx_hbm, z_hbm, out_hbm, idxs_v, rows_v, acc_sh):
        c = lax.axis_index("c")
        s = lax.axis_index("s")
        wid = s * 2 + c
        pltpu.sync_copy(z_hbm, acc_sh.at[pl.ds(s * ROWS_PER_TILE, ROWS_PER_TILE)])
        pltpu.sync_copy(idx_hbm.at[pl.ds(wid * C_PER_W, C_PER_W)], idxs_v)
        plsc.subcore_barrier()

        def body(j, carry):
            pltpu.sync_copy(
                rows_hbm.at[pl.ds((wid * C_PER_W + j) * CHUNK, CHUNK)], rows_v)
            pltpu.sync_copy(rows_v, acc_sh.at[idxs_v.at[j]], add=True)
            return carry

        lax.fori_loop(0, C_PER_W, body, 0)
        plsc.subcore_barrier()
        pltpu.sync_copy(
            acc_sh.at[pl.ds(s * ROWS_PER_TILE, ROWS_PER_TILE)],
            out_hbm.at[c, pl.ds(s * ROWS_PER_TILE, ROWS_PER_TILE)])

    return k


def _sc_scatter_add(rows, idx2, W):
    zrows = jnp.zeros((ROWS_PER_TILE, W), _f32)
    return _sc_scatter_kernel(W)(rows, idx2, zrows)


def _sc_degree(idx2):
    mesh = plsc.VectorSubcoreMesh(core_axis_name="c", subcore_axis_name="s")
    Wd = 16
    zrows = jnp.zeros((ROWS_PER_TILE, Wd), _f32)
    ones = jnp.ones((CHUNK, Wd), _f32)

    @functools.partial(
        pl.kernel, mesh=mesh,
        out_type=jax.ShapeDtypeStruct((2, N_SP, Wd), _f32),
        compiler_params=pltpu.CompilerParams(use_tc_tiling_on_sc=False),
        scratch_types=[
            pltpu.VMEM((C_PER_W, CHUNK), jnp.int32),
            pltpu.VMEM((CHUNK, Wd), _f32),
            pltpu.VMEM_SHARED((N_SP, Wd), _f32),
        ],
    )
    def k(idx_hbm, z_hbm, ones_hbm, out_hbm, idxs_v, ones_v, acc_sh):
        c = lax.axis_index("c")
        s = lax.axis_index("s")
        wid = s * 2 + c
        pltpu.sync_copy(z_hbm, acc_sh.at[pl.ds(s * ROWS_PER_TILE, ROWS_PER_TILE)])
        pltpu.sync_copy(idx_hbm.at[pl.ds(wid * C_PER_W, C_PER_W)], idxs_v)
        pltpu.sync_copy(ones_hbm, ones_v)
        plsc.subcore_barrier()

        def body(j, carry):
            pltpu.sync_copy(ones_v, acc_sh.at[idxs_v.at[j]], add=True)
            return carry

        lax.fori_loop(0, C_PER_W, body, 0)
        plsc.subcore_barrier()
        pltpu.sync_copy(
            acc_sh.at[pl.ds(s * ROWS_PER_TILE, ROWS_PER_TILE)],
            out_hbm.at[c, pl.ds(s * ROWS_PER_TILE, ROWS_PER_TILE)])

    return k(idx2, zrows, ones)


BN = 1000
BE = 640


def _relu(x):
    return jnp.maximum(x, 0.0)


def _lin0_body(nf, ws, bs, wu, bu, o):
    x = nf[...]
    a = _relu(jnp.dot(x, ws[...], preferred_element_type=_f32) + bs[...])
    b = _relu(jnp.dot(x, wu[...], preferred_element_type=_f32) + bu[...])
    o[...] = jnp.concatenate([a, b], axis=1)


def _tc_lin0(nfeat, ps, pu):
    return pl.pallas_call(
        _lin0_body,
        grid=(N // BN,),
        in_specs=[
            pl.BlockSpec((BN, F_IN), lambda i: (i, 0)),
            pl.BlockSpec((F_IN, HID), lambda i: (0, 0)),
            pl.BlockSpec((1, HID), lambda i: (0, 0)),
            pl.BlockSpec((F_IN, HID), lambda i: (0, 0)),
            pl.BlockSpec((1, HID), lambda i: (0, 0)),
        ],
        out_specs=pl.BlockSpec((BN, 2 * HID), lambda i: (i, 0)),
        out_shape=jax.ShapeDtypeStruct((N, 2 * HID), _f32),
    )(nfeat, ps['lin0_w'], ps['lin0_b'].reshape(1, -1),
      pu['lin0_w'], pu['lin0_b'].reshape(1, -1))


def _edgenet_body(ef, w1s, b1s, w2s, b2s, w1u, b1u, w2u, b2u, os_, ou_):
    x = ef[...]
    hs = _relu(jnp.dot(x, w1s[...], preferred_element_type=_f32) + b1s[...])
    os_[...] = jnp.dot(hs, w2s[...], preferred_element_type=_f32) + b2s[...]
    hu = _relu(jnp.dot(x, w1u[...], preferred_element_type=_f32) + b1u[...])
    ou_[...] = jnp.dot(hu, w2u[...], preferred_element_type=_f32) + b2u[...]


def _tc_edgenet(efeat_pad, ps, pu):
    HH = HID * HID
    return pl.pallas_call(
        _edgenet_body,
        grid=(E_PAD // BE,),
        in_specs=[
            pl.BlockSpec((BE, 5), lambda i: (i, 0)),
            pl.BlockSpec((5, 128), lambda i: (0, 0)),
            pl.BlockSpec((1, 128), lambda i: (0, 0)),
            pl.BlockSpec((128, HH), lambda i: (0, 0)),
            pl.BlockSpec((1, HH), lambda i: (0, 0)),
            pl.BlockSpec((5, 128), lambda i: (0, 0)),
            pl.BlockSpec((1, 128), lambda i: (0, 0)),
            pl.BlockSpec((128, HH), lambda i: (0, 0)),
            pl.BlockSpec((1, HH), lambda i: (0, 0)),
        ],
        out_specs=[
            pl.BlockSpec((BE, HH), lambda i: (i, 0)),
            pl.BlockSpec((BE, HH), lambda i: (i, 0)),
        ],
        out_shape=[
            jax.ShapeDtypeStruct((E_PAD, HH), _f32),
            jax.ShapeDtypeStruct((E_PAD, HH), _f32),
        ],
    )(efeat_pad, ps['nn_w1'], ps['nn_b1'].reshape(1, -1),
      ps['nn_w2'], ps['nn_b2'].reshape(1, -1),
      pu['nn_w1'], pu['nn_b1'].reshape(1, -1),
      pu['nn_w2'], pu['nn_b2'].reshape(1, -1))


def _matvec_body(osrc, ews, ewu, o):
    x = osrc[...]
    accs = jnp.zeros((BE, HID), _f32)
    accu = jnp.zeros((BE, HID), _f32)
    for h in range(HID):
        accs = accs + x[:, h:h + 1] * ews[:, h * HID:(h + 1) * HID]
        accu = accu + x[:, HID + h:HID + h + 1] * ewu[:, h * HID:(h + 1) * HID]
    o[...] = jnp.concatenate([accs, accu], axis=1)


def _tc_matvec(out_src, ew_s, ew_u):
    HH = HID * HID
    return pl.pallas_call(
        _matvec_body,
        grid=(E_PAD // BE,),
        in_specs=[
            pl.BlockSpec((BE, 2 * HID), lambda i: (i, 0)),
            pl.BlockSpec((BE, HH), lambda i: (i, 0)),
            pl.BlockSpec((BE, HH), lambda i: (i, 0)),
        ],
        out_specs=pl.BlockSpec((BE, 2 * HID), lambda i: (i, 0)),
        out_shape=jax.ShapeDtypeStruct((E_PAD, 2 * HID), _f32),
    )(out_src, ew_s, ew_u)


def _gru_half(m, h, wi, bi, wh, bh):
    gi = jnp.dot(m, wi, preferred_element_type=_f32) + bi
    gh = jnp.dot(h, wh, preferred_element_type=_f32) + bh
    r = jax.nn.sigmoid(gi[:, :HID] + gh[:, :HID])
    z = jax.nn.sigmoid(gi[:, HID:2 * HID] + gh[:, HID:2 * HID])
    n = jnp.tanh(gi[:, 2 * HID:] + r * gh[:, 2 * HID:])
    return (1.0 - z) * n + z * h


def _gru_body(mp, dp, hc, cbs, cbu, wis, bis, whs, bhs, wiu, biu, whu, bhu, o):
    deg = dp[0][:, 0:1] + dp[1][:, 0:1]
    rdeg = 1.0 / jnp.maximum(deg, 1.0)
    msum = (mp[0] + mp[1]) * rdeg
    m_s = _relu(msum[:, :HID] + cbs[...])
    m_u = _relu(msum[:, HID:] + cbu[...])
    h_s = hc[...][:, :HID]
    h_u = hc[...][:, HID:]
    ns = _gru_half(m_s, h_s, wis[...], bis[...], whs[...], bhs[...])
    nu = _gru_half(m_u, h_u, wiu[...], biu[...], whu[...], bhu[...])
    o[...] = jnp.concatenate([ns, nu], axis=1)


def _tc_gru(mp, degp, hc, ps, pu):
    return pl.pallas_call(
        _gru_body,
        grid=(N // BN,),
        in_specs=[
            pl.BlockSpec((2, BN, 2 * HID), lambda i: (0, i, 0)),
            pl.BlockSpec((2, BN, 16), lambda i: (0, i, 0)),
            pl.BlockSpec((BN, 2 * HID), lambda i: (i, 0)),
            pl.BlockSpec((1, HID), lambda i: (0, 0)),
            pl.BlockSpec((1, HID), lambda i: (0, 0)),
            pl.BlockSpec((HID, 3 * HID), lambda i: (0, 0)),
            pl.BlockSpec((1, 3 * HID), lambda i: (0, 0)),
            pl.BlockSpec((HID, 3 * HID), lambda i: (0, 0)),
            pl.BlockSpec((1, 3 * HID), lambda i: (0, 0)),
            pl.BlockSpec((HID, 3 * HID), lambda i: (0, 0)),
            pl.BlockSpec((1, 3 * HID), lambda i: (0, 0)),
            pl.BlockSpec((HID, 3 * HID), lambda i: (0, 0)),
            pl.BlockSpec((1, 3 * HID), lambda i: (0, 0)),
        ],
        out_specs=pl.BlockSpec((BN, 2 * HID), lambda i: (i, 0)),
        out_shape=jax.ShapeDtypeStruct((N, 2 * HID), _f32),
    )(mp, degp, hc,
      ps['conv_b'].reshape(1, -1), pu['conv_b'].reshape(1, -1),
      ps['gru_wi'], ps['gru_bi'].reshape(1, -1),
      ps['gru_wh'], ps['gru_bh'].reshape(1, -1),
      pu['gru_wi'], pu['gru_bi'].reshape(1, -1),
      pu['gru_wh'], pu['gru_bh'].reshape(1, -1))


def _softplus(x):
    return jnp.maximum(x, 0.0) + jnp.log(1.0 + jnp.exp(-jnp.abs(x)))


def _pos_exp(x):
    return LOG2 - _softplus(-x)


def _neg_exp(x):
    return _softplus(-x) + x - LOG2


def _ffnn_in(x, w1, b1, w2, b2, w3, b3, jw, jb):
    h = _relu(jnp.dot(x, w1, preferred_element_type=_f32) + b1)
    h = _relu(jnp.dot(h, w2, preferred_element_type=_f32) + b2)
    h = _relu(jnp.dot(h, w3, preferred_element_type=_f32) + b3)
    return h + jnp.dot(x, jw, preferred_element_type=_f32) + jb


def _set2set(out, oh, wi, bi, wh, bh):
    q_star = jnp.zeros((G, 2 * HID), _f32)
    hh = jnp.zeros((G, HID), _f32)
    cc = jnp.zeros((G, HID), _f32)
    for _ in range(3):
        gates = (jnp.dot(q_star, wi, preferred_element_type=_f32) + bi
                 + jnp.dot(hh, wh, preferred_element_type=_f32) + bh)
        i_, f_, g_, o_ = (gates[:, :HID], gates[:, HID:2 * HID],
                          gates[:, 2 * HID:3 * HID], gates[:, 3 * HID:])
        cc = jax.nn.sigmoid(f_) * cc + jax.nn.sigmoid(i_) * jnp.tanh(g_)
        hh = jax.nn.sigmoid(o_) * jnp.tanh(cc)
        qn = jnp.dot(oh, hh, preferred_element_type=_f32)
        e = jnp.sum(out * qn, axis=1, keepdims=True)
        em = jnp.max(jnp.where(oh > 0.0, e, -1e30), axis=0, keepdims=True)
        ee = jnp.exp(e - jnp.dot(oh, em.T, preferred_element_type=_f32))
        denom = lax.dot_general(oh, ee, (((0,), (0,)), ((), ())),
                                preferred_element_type=_f32)
        inv = 1.0 / jnp.maximum(denom, 1e-30)
        a = ee * jnp.dot(oh, inv, preferred_element_type=_f32)
        r = lax.dot_general(oh, a * out, (((0,), (0,)), ((), ())),
                            preferred_element_type=_f32)
        q_star = jnp.concatenate([hh, r], axis=1)
    return q_star


def _final_body(hc, gid, s2s_s, s2s_u, fc, f_ugd, f_uld, f_sd, f_ud,
                pred_o, ul_o, cl_o):
    out_s = hc[...][:, :HID]
    out_u = hc[...][:, HID:]
    g = gid[...]
    oh = (lax.broadcasted_iota(jnp.int32, (N, G), 1) == g).astype(_f32)

    sup_emb = _set2set(out_s, oh, *[r[...] for r in s2s_s])
    unsup_emb = _set2set(out_u, oh, *[r[...] for r in s2s_u])

    fc1_w, fc1_b, fc2_w, fc2_b = [r[...] for r in fc]
    pred = (jnp.dot(_relu(jnp.dot(sup_emb, fc1_w,
                                  preferred_element_type=_f32) + fc1_b),
                    fc2_w, preferred_element_type=_f32) + fc2_b)
    pred_o[...] = pred

    g_enc = _ffnn_in(unsup_emb, *[r[...] for r in f_ugd])
    l_enc = _ffnn_in(out_u, *[r[...] for r in f_uld])
    sup_g = _ffnn_in(sup_emb, *[r[...] for r in f_sd])
    unsup_g = _ffnn_in(unsup_emb, *[r[...] for r in f_ud])

    res = lax.dot_general(l_enc, g_enc, (((1,), (1,)), ((), ())),
                          preferred_element_type=_f32)
    e_pos = jnp.sum(_pos_exp(res * oh)) / N
    e_neg = jnp.sum(_neg_exp(res * (1.0 - oh))) / (N * (G - 1))
    ul_o[...] = jnp.broadcast_to(e_neg - e_pos, (1, 1))

    res2 = lax.dot_general(sup_g, unsup_g, (((1,), (1,)), ((), ())),
                           preferred_element_type=_f32)
    eye = (lax.broadcasted_iota(jnp.int32, (G, G), 0)
           == lax.broadcasted_iota(jnp.int32, (G, G), 1)).astype(_f32)
    e_pos2 = jnp.sum(_pos_exp(res2 * eye)) / G
    e_neg2 = jnp.sum(_neg_exp(res2 * (1.0 - eye))) / (G * (G - 1))
    cl_o[...] = jnp.broadcast_to(e_neg2 - e_pos2, (1, 1))


def _tc_final(hc, gid2d, params):
    ps, pu = params['sup'], params['unsup']
    s2s_s = [ps['s2s_wi'], ps['s2s_bi'].reshape(1, -1),
             ps['s2s_wh'], ps['s2s_bh'].reshape(1, -1)]
    s2s_u = [pu['s2s_wi'], pu['s2s_bi'].reshape(1, -1),
             pu['s2s_wh'], pu['s2s_bh'].reshape(1, -1)]
    fc = [params['fc1_w'], params['fc1_b'].reshape(1, -1),
          params['fc2_w'], params['fc2_b'].reshape(1, -1)]

    def ffnn_list(p):
        return [p['w1'], p['b1'].reshape(1, -1), p['w2'], p['b2'].reshape(1, -1),
                p['w3'], p['b3'].reshape(1, -1), p['jw'], p['jb'].reshape(1, -1)]

    full = lambda x: pl.BlockSpec(x.shape, lambda: tuple(0 for _ in x.shape))
    args = [hc, gid2d] + s2s_s + s2s_u + fc + \
        ffnn_list(params['ugd']) + ffnn_list(params['uld']) + \
        ffnn_list(params['sd']) + ffnn_list(params['ud'])

    def body(*refs):
        hc_r, gid_r = refs[0], refs[1]
        s2ss = refs[2:6]
        s2su = refs[6:10]
        fcr = refs[10:14]
        ugd = refs[14:22]
        uld = refs[22:30]
        sd = refs[30:38]
        ud = refs[38:46]
        pred_o, ul_o, cl_o = refs[46], refs[47], refs[48]
        _final_body(hc_r, gid_r, s2ss, s2su, fcr, ugd, uld, sd, ud,
                    pred_o, ul_o, cl_o)

    return pl.pallas_call(
        body,
        in_specs=[full(a) for a in args],
        out_specs=[
            pl.BlockSpec((G, 1), lambda: (0, 0)),
            pl.BlockSpec((1, 1), lambda: (0, 0)),
            pl.BlockSpec((1, 1), lambda: (0, 0)),
        ],
        out_shape=[
            jax.ShapeDtypeStruct((G, 1), _f32),
            jax.ShapeDtypeStruct((1, 1), _f32),
            jax.ShapeDtypeStruct((1, 1), _f32),
        ],
    )(*args)


def kernel(nfeat, efeat, edge_index, graph_id, params):
    ps, pu = params['sup'], params['unsup']
    src = edge_index[0].astype(jnp.int32)
    dst = edge_index[1].astype(jnp.int32)
    pad = E_PAD - E
    src2 = jnp.concatenate([src, jnp.zeros((pad,), jnp.int32)]).reshape(-1, CHUNK)
    dst2 = jnp.concatenate([dst, jnp.full((pad,), N, jnp.int32)]).reshape(-1, CHUNK)
    efeat_pad = jnp.concatenate([efeat, jnp.zeros((pad, 5), _f32)], axis=0)
    gid2d = graph_id.astype(jnp.int32).reshape(N, 1)

    degp = _sc_degree(dst2)
    hc = _tc_lin0(nfeat, ps, pu)
    ew_s, ew_u = _tc_edgenet(efeat_pad, ps, pu)

    for _ in range(3):
        out_src = _sc_gather(hc, src2)
        msg = _tc_matvec(out_src, ew_s, ew_u)
        mp = _sc_scatter_add(msg, dst2, 2 * HID)
        hc = _tc_gru(mp[:, :N, :], degp[:, :N, :], hc, ps, pu)

    pred, ul, cl = _tc_final(hc, gid2d, params)
    return pred.reshape(-1), ul.reshape(()), cl.reshape(())


# speedup vs baseline: 1.0421x; 1.0421x over previous
import functools
import math

import jax
import jax.numpy as jnp
from jax import lax
from jax.experimental import pallas as pl
from jax.experimental.pallas import tpu as pltpu
from jax.experimental.pallas import tpu_sc as plsc

N = 10000
E = 160000
F_IN = 128
HID = 32
G = 64
LOG2 = math.log(2.0)

NW = 32
CHUNK = 128
C_PER_W = 40
E_PAD = NW * CHUNK * C_PER_W
N_SP = 10240
ROWS_PER_TILE = N_SP // 16

_f32 = jnp.float32


@functools.lru_cache(maxsize=None)
def _sc_gather_kernel(W):
    mesh = plsc.VectorSubcoreMesh(core_axis_name="c", subcore_axis_name="s")

    @functools.partial(
        pl.kernel, mesh=mesh,
        out_type=jax.ShapeDtypeStruct((E_PAD, W), _f32),
        compiler_params=pltpu.CompilerParams(use_tc_tiling_on_sc=False),
        scratch_types=[
            pltpu.VMEM((C_PER_W, CHUNK), jnp.int32),
            pltpu.VMEM((CHUNK, W), _f32),
            pltpu.SemaphoreType.DMA,
        ],
    )
    def k(table_hbm, idx_hbm, out_hbm, idxs_v, rows_v, sem):
        c = lax.axis_index("c")
        s = lax.axis_index("s")
        wid = s * 2 + c
        pltpu.sync_copy(idx_hbm.at[pl.ds(wid * C_PER_W, C_PER_W)], idxs_v)

        def body(j, carry):
            pltpu.async_copy(table_hbm.at[idxs_v.at[j]], rows_v, sem).wait()
            pltpu.sync_copy(rows_v, out_hbm.at[pl.ds((wid * C_PER_W + j) * CHUNK, CHUNK)])
            return carry

        lax.fori_loop(0, C_PER_W, body, 0)

    return k


def _sc_gather(table, idx2):
    return _sc_gather_kernel(table.shape[1])(table, idx2)


@functools.lru_cache(maxsize=None)
def _sc_scatter_kernel(W):
    mesh = plsc.VectorSubcoreMesh(core_axis_name="c", subcore_axis_name="s")

    @functools.partial(
        pl.kernel, mesh=mesh,
        out_type=jax.ShapeDtypeStruct((2, N_SP, W), _f32),
        compiler_params=pltpu.CompilerParams(use_tc_tiling_on_sc=False),
        scratch_types=[
            pltpu.VMEM((C_PER_W, CHUNK), jnp.int32),
            pltpu.VMEM((CHUNK, W), _f32),
            pltpu.VMEM_SHARED((N_SP, W), _f32),
        ],
    )
    def k(rows_hbm, idx_hbm, z_hbm, out_hbm, idxs_v, rows_v, acc_sh):
        c = lax.axis_index("c")
        s = lax.axis_index("s")
        wid = s * 2 + c
        pltpu.sync_copy(z_hbm, acc_sh.at[pl.ds(s * ROWS_PER_TILE, ROWS_PER_TILE)])
        pltpu.sync_copy(idx_hbm.at[pl.ds(wid * C_PER_W, C_PER_W)], idxs_v)
        plsc.subcore_barrier()

        def body(j, carry):
            pltpu.sync_copy(
                rows_hbm.at[pl.ds((wid * C_PER_W + j) * CHUNK, CHUNK)], rows_v)
            pltpu.sync_copy(rows_v, acc_sh.at[idxs_v.at[j]], add=True)
            return carry

        lax.fori_loop(0, C_PER_W, body, 0)
        plsc.subcore_barrier()
        pltpu.sync_copy(
            acc_sh.at[pl.ds(s * ROWS_PER_TILE, ROWS_PER_TILE)],
            out_hbm.at[c, pl.ds(s * ROWS_PER_TILE, ROWS_PER_TILE)])

    return k


def _sc_scatter_add(rows, idx2, W):
    zrows = jnp.zeros((ROWS_PER_TILE, W), _f32)
    return _sc_scatter_kernel(W)(rows, idx2, zrows)


def _sc_degree(idx2):
    mesh = plsc.VectorSubcoreMesh(core_axis_name="c", subcore_axis_name="s")
    Wd = 16
    zrows = jnp.zeros((ROWS_PER_TILE, Wd), _f32)
    ones = jnp.ones((CHUNK, Wd), _f32)

    @functools.partial(
        pl.kernel, mesh=mesh,
        out_type=jax.ShapeDtypeStruct((2, N_SP, Wd), _f32),
        compiler_params=pltpu.CompilerParams(use_tc_tiling_on_sc=False),
        scratch_types=[
            pltpu.VMEM((C_PER_W, CHUNK), jnp.int32),
            pltpu.VMEM((CHUNK, Wd), _f32),
            pltpu.VMEM_SHARED((N_SP, Wd), _f32),
        ],
    )
    def k(idx_hbm, z_hbm, ones_hbm, out_hbm, idxs_v, ones_v, acc_sh):
        c = lax.axis_index("c")
        s = lax.axis_index("s")
        wid = s * 2 + c
        pltpu.sync_copy(z_hbm, acc_sh.at[pl.ds(s * ROWS_PER_TILE, ROWS_PER_TILE)])
        pltpu.sync_copy(idx_hbm.at[pl.ds(wid * C_PER_W, C_PER_W)], idxs_v)
        pltpu.sync_copy(ones_hbm, ones_v)
        plsc.subcore_barrier()

        def body(j, carry):
            pltpu.sync_copy(ones_v, acc_sh.at[idxs_v.at[j]], add=True)
            return carry

        lax.fori_loop(0, C_PER_W, body, 0)
        plsc.subcore_barrier()
        pltpu.sync_copy(
            acc_sh.at[pl.ds(s * ROWS_PER_TILE, ROWS_PER_TILE)],
            out_hbm.at[c, pl.ds(s * ROWS_PER_TILE, ROWS_PER_TILE)])

    return k(idx2, zrows, ones)


BN = 1000
BE = 640


def _relu(x):
    return jnp.maximum(x, 0.0)


def _lin0_body(nf, ws, bs, wu, bu, o):
    x = nf[...]
    a = _relu(jnp.dot(x, ws[...], preferred_element_type=_f32) + bs[...])
    b = _relu(jnp.dot(x, wu[...], preferred_element_type=_f32) + bu[...])
    o[...] = jnp.concatenate([a, b], axis=1)


def _tc_lin0(nfeat, ps, pu):
    return pl.pallas_call(
        _lin0_body,
        grid=(N // BN,),
        in_specs=[
            pl.BlockSpec((BN, F_IN), lambda i: (i, 0)),
            pl.BlockSpec((F_IN, HID), lambda i: (0, 0)),
            pl.BlockSpec((1, HID), lambda i: (0, 0)),
            pl.BlockSpec((F_IN, HID), lambda i: (0, 0)),
            pl.BlockSpec((1, HID), lambda i: (0, 0)),
        ],
        out_specs=pl.BlockSpec((BN, 2 * HID), lambda i: (i, 0)),
        out_shape=jax.ShapeDtypeStruct((N, 2 * HID), _f32),
    )(nfeat, ps['lin0_w'], ps['lin0_b'].reshape(1, -1),
      pu['lin0_w'], pu['lin0_b'].reshape(1, -1))


def _edgenet_body(ef, w1s, b1s, w2s, b2s, w1u, b1u, w2u, b2u, os_, ou_):
    x = ef[...]
    hs = _relu(jnp.dot(x, w1s[...], preferred_element_type=_f32) + b1s[...])
    os_[...] = jnp.dot(hs, w2s[...], preferred_element_type=_f32) + b2s[...]
    hu = _relu(jnp.dot(x, w1u[...], preferred_element_type=_f32) + b1u[...])
    ou_[...] = jnp.dot(hu, w2u[...], preferred_element_type=_f32) + b2u[...]


def _tc_edgenet(efeat_pad, ps, pu):
    HH = HID * HID
    return pl.pallas_call(
        _edgenet_body,
        grid=(E_PAD // BE,),
        in_specs=[
            pl.BlockSpec((BE, 5), lambda i: (i, 0)),
            pl.BlockSpec((5, 128), lambda i: (0, 0)),
            pl.BlockSpec((1, 128), lambda i: (0, 0)),
            pl.BlockSpec((128, HH), lambda i: (0, 0)),
            pl.BlockSpec((1, HH), lambda i: (0, 0)),
            pl.BlockSpec((5, 128), lambda i: (0, 0)),
            pl.BlockSpec((1, 128), lambda i: (0, 0)),
            pl.BlockSpec((128, HH), lambda i: (0, 0)),
            pl.BlockSpec((1, HH), lambda i: (0, 0)),
        ],
        out_specs=[
            pl.BlockSpec((BE, HH), lambda i: (i, 0)),
            pl.BlockSpec((BE, HH), lambda i: (i, 0)),
        ],
        out_shape=[
            jax.ShapeDtypeStruct((E_PAD, HH), _f32),
            jax.ShapeDtypeStruct((E_PAD, HH), _f32),
        ],
    )(efeat_pad, ps['nn_w1'], ps['nn_b1'].reshape(1, -1),
      ps['nn_w2'], ps['nn_b2'].reshape(1, -1),
      pu['nn_w1'], pu['nn_b1'].reshape(1, -1),
      pu['nn_w2'], pu['nn_b2'].reshape(1, -1))


def _matvec_body(osrc, ef, w1s, b1s, w2s, b2s, w1u, b1u, w2u, b2u, o):
    x = osrc[...]
    e = ef[...]
    hs = _relu(jnp.dot(e, w1s[...], preferred_element_type=_f32) + b1s[...])
    ews = jnp.dot(hs, w2s[...], preferred_element_type=_f32) + b2s[...]
    hu = _relu(jnp.dot(e, w1u[...], preferred_element_type=_f32) + b1u[...])
    ewu = jnp.dot(hu, w2u[...], preferred_element_type=_f32) + b2u[...]
    accs = jnp.zeros((BE, HID), _f32)
    accu = jnp.zeros((BE, HID), _f32)
    for h in range(HID):
        accs = accs + x[:, h:h + 1] * ews[:, h * HID:(h + 1) * HID]
        accu = accu + x[:, HID + h:HID + h + 1] * ewu[:, h * HID:(h + 1) * HID]
    o[...] = jnp.concatenate([accs, accu], axis=1)


def _tc_matvec(out_src, efeat_pad, ps, pu):
    HH = HID * HID
    return pl.pallas_call(
        _matvec_body,
        grid=(E_PAD // BE,),
        in_specs=[
            pl.BlockSpec((BE, 2 * HID), lambda i: (i, 0)),
            pl.BlockSpec((BE, 5), lambda i: (i, 0)),
            pl.BlockSpec((5, 128), lambda i: (0, 0)),
            pl.BlockSpec((1, 128), lambda i: (0, 0)),
            pl.BlockSpec((128, HH), lambda i: (0, 0)),
            pl.BlockSpec((1, HH), lambda i: (0, 0)),
            pl.BlockSpec((5, 128), lambda i: (0, 0)),
            pl.BlockSpec((1, 128), lambda i: (0, 0)),
            pl.BlockSpec((128, HH), lambda i: (0, 0)),
            pl.BlockSpec((1, HH), lambda i: (0, 0)),
        ],
        out_specs=pl.BlockSpec((BE, 2 * HID), lambda i: (i, 0)),
        out_shape=jax.ShapeDtypeStruct((E_PAD, 2 * HID), _f32),
    )(out_src, efeat_pad,
      ps['nn_w1'], ps['nn_b1'].reshape(1, -1),
      ps['nn_w2'], ps['nn_b2'].reshape(1, -1),
      pu['nn_w1'], pu['nn_b1'].reshape(1, -1),
      pu['nn_w2'], pu['nn_b2'].reshape(1, -1))


def _gru_half(m, h, wi, bi, wh, bh):
    gi = jnp.dot(m, wi, preferred_element_type=_f32) + bi
    gh = jnp.dot(h, wh, preferred_element_type=_f32) + bh
    r = jax.nn.sigmoid(gi[:, :HID] + gh[:, :HID])
    z = jax.nn.sigmoid(gi[:, HID:2 * HID] + gh[:, HID:2 * HID])
    n = jnp.tanh(gi[:, 2 * HID:] + r * gh[:, 2 * HID:])
    return (1.0 - z) * n + z * h


def _gru_body(mp, dp, hc, cbs, cbu, wis, bis, whs, bhs, wiu, biu, whu, bhu, o):
    deg = dp[0][:, 0:1] + dp[1][:, 0:1]
    rdeg = 1.0 / jnp.maximum(deg, 1.0)
    msum = (mp[0] + mp[1]) * rdeg
    m_s = _relu(msum[:, :HID] + cbs[...])
    m_u = _relu(msum[:, HID:] + cbu[...])
    h_s = hc[...][:, :HID]
    h_u = hc[...][:, HID:]
    ns = _gru_half(m_s, h_s, wis[...], bis[...], whs[...], bhs[...])
    nu = _gru_half(m_u, h_u, wiu[...], biu[...], whu[...], bhu[...])
    o[...] = jnp.concatenate([ns, nu], axis=1)


def _tc_gru(mp, degp, hc, ps, pu):
    return pl.pallas_call(
        _gru_body,
        grid=(N // BN,),
        in_specs=[
            pl.BlockSpec((2, BN, 2 * HID), lambda i: (0, i, 0)),
            pl.BlockSpec((2, BN, 16), lambda i: (0, i, 0)),
            pl.BlockSpec((BN, 2 * HID), lambda i: (i, 0)),
            pl.BlockSpec((1, HID), lambda i: (0, 0)),
            pl.BlockSpec((1, HID), lambda i: (0, 0)),
            pl.BlockSpec((HID, 3 * HID), lambda i: (0, 0)),
            pl.BlockSpec((1, 3 * HID), lambda i: (0, 0)),
            pl.BlockSpec((HID, 3 * HID), lambda i: (0, 0)),
            pl.BlockSpec((1, 3 * HID), lambda i: (0, 0)),
            pl.BlockSpec((HID, 3 * HID), lambda i: (0, 0)),
            pl.BlockSpec((1, 3 * HID), lambda i: (0, 0)),
            pl.BlockSpec((HID, 3 * HID), lambda i: (0, 0)),
            pl.BlockSpec((1, 3 * HID), lambda i: (0, 0)),
        ],
        out_specs=pl.BlockSpec((BN, 2 * HID), lambda i: (i, 0)),
        out_shape=jax.ShapeDtypeStruct((N, 2 * HID), _f32),
    )(mp, degp, hc,
      ps['conv_b'].reshape(1, -1), pu['conv_b'].reshape(1, -1),
      ps['gru_wi'], ps['gru_bi'].reshape(1, -1),
      ps['gru_wh'], ps['gru_bh'].reshape(1, -1),
      pu['gru_wi'], pu['gru_bi'].reshape(1, -1),
      pu['gru_wh'], pu['gru_bh'].reshape(1, -1))


def _softplus(x):
    return jnp.maximum(x, 0.0) + jnp.log(1.0 + jnp.exp(-jnp.abs(x)))


def _pos_exp(x):
    return LOG2 - _softplus(-x)


def _neg_exp(x):
    return _softplus(-x) + x - LOG2


def _ffnn_in(x, w1, b1, w2, b2, w3, b3, jw, jb):
    h = _relu(jnp.dot(x, w1, preferred_element_type=_f32) + b1)
    h = _relu(jnp.dot(h, w2, preferred_element_type=_f32) + b2)
    h = _relu(jnp.dot(h, w3, preferred_element_type=_f32) + b3)
    return h + jnp.dot(x, jw, preferred_element_type=_f32) + jb


def _set2set(out, oh, wi, bi, wh, bh):
    q_star = jnp.zeros((G, 2 * HID), _f32)
    hh = jnp.zeros((G, HID), _f32)
    cc = jnp.zeros((G, HID), _f32)
    for _ in range(3):
        gates = (jnp.dot(q_star, wi, preferred_element_type=_f32) + bi
                 + jnp.dot(hh, wh, preferred_element_type=_f32) + bh)
        i_, f_, g_, o_ = (gates[:, :HID], gates[:, HID:2 * HID],
                          gates[:, 2 * HID:3 * HID], gates[:, 3 * HID:])
        cc = jax.nn.sigmoid(f_) * cc + jax.nn.sigmoid(i_) * jnp.tanh(g_)
        hh = jax.nn.sigmoid(o_) * jnp.tanh(cc)
        qn = jnp.dot(oh, hh, preferred_element_type=_f32)
        e = jnp.sum(out * qn, axis=1, keepdims=True)
        em = jnp.max(jnp.where(oh > 0.0, e, -1e30), axis=0, keepdims=True)
        ee = jnp.exp(e - jnp.dot(oh, em.T, preferred_element_type=_f32))
        denom = lax.dot_general(oh, ee, (((0,), (0,)), ((), ())),
                                preferred_element_type=_f32)
        inv = 1.0 / jnp.maximum(denom, 1e-30)
        a = ee * jnp.dot(oh, inv, preferred_element_type=_f32)
        r = lax.dot_general(oh, a * out, (((0,), (0,)), ((), ())),
                            preferred_element_type=_f32)
        q_star = jnp.concatenate([hh, r], axis=1)
    return q_star


def _final_body(hc, gid, s2s_s, s2s_u, fc, f_ugd, f_uld, f_sd, f_ud,
                pred_o, ul_o, cl_o):
    out_s = hc[...][:, :HID]
    out_u = hc[...][:, HID:]
    g = gid[...]
    oh = (lax.broadcasted_iota(jnp.int32, (N, G), 1) == g).astype(_f32)

    sup_emb = _set2set(out_s, oh, *[r[...] for r in s2s_s])
    unsup_emb = _set2set(out_u, oh, *[r[...] for r in s2s_u])

    fc1_w, fc1_b, fc2_w, fc2_b = [r[...] for r in fc]
    pred = (jnp.dot(_relu(jnp.dot(sup_emb, fc1_w,
                                  preferred_element_type=_f32) + fc1_b),
                    fc2_w, preferred_element_type=_f32) + fc2_b)
    pred_o[...] = pred

    g_enc = _ffnn_in(unsup_emb, *[r[...] for r in f_ugd])
    l_enc = _ffnn_in(out_u, *[r[...] for r in f_uld])
    sup_g = _ffnn_in(sup_emb, *[r[...] for r in f_sd])
    unsup_g = _ffnn_in(unsup_emb, *[r[...] for r in f_ud])

    res = lax.dot_general(l_enc, g_enc, (((1,), (1,)), ((), ())),
                          preferred_element_type=_f32)
    e_pos = jnp.sum(_pos_exp(res * oh)) / N
    e_neg = jnp.sum(_neg_exp(res * (1.0 - oh))) / (N * (G - 1))
    ul_o[...] = jnp.broadcast_to(e_neg - e_pos, (1, 1))

    res2 = lax.dot_general(sup_g, unsup_g, (((1,), (1,)), ((), ())),
                           preferred_element_type=_f32)
    eye = (lax.broadcasted_iota(jnp.int32, (G, G), 0)
           == lax.broadcasted_iota(jnp.int32, (G, G), 1)).astype(_f32)
    e_pos2 = jnp.sum(_pos_exp(res2 * eye)) / G
    e_neg2 = jnp.sum(_neg_exp(res2 * (1.0 - eye))) / (G * (G - 1))
    cl_o[...] = jnp.broadcast_to(e_neg2 - e_pos2, (1, 1))


def _tc_final(hc, gid2d, params):
    ps, pu = params['sup'], params['unsup']
    s2s_s = [ps['s2s_wi'], ps['s2s_bi'].reshape(1, -1),
             ps['s2s_wh'], ps['s2s_bh'].reshape(1, -1)]
    s2s_u = [pu['s2s_wi'], pu['s2s_bi'].reshape(1, -1),
             pu['s2s_wh'], pu['s2s_bh'].reshape(1, -1)]
    fc = [params['fc1_w'], params['fc1_b'].reshape(1, -1),
          params['fc2_w'], params['fc2_b'].reshape(1, -1)]

    def ffnn_list(p):
        return [p['w1'], p['b1'].reshape(1, -1), p['w2'], p['b2'].reshape(1, -1),
                p['w3'], p['b3'].reshape(1, -1), p['jw'], p['jb'].reshape(1, -1)]

    full = lambda x: pl.BlockSpec(x.shape, lambda: tuple(0 for _ in x.shape))
    args = [hc, gid2d] + s2s_s + s2s_u + fc + \
        ffnn_list(params['ugd']) + ffnn_list(params['uld']) + \
        ffnn_list(params['sd']) + ffnn_list(params['ud'])

    def body(*refs):
        hc_r, gid_r = refs[0], refs[1]
        s2ss = refs[2:6]
        s2su = refs[6:10]
        fcr = refs[10:14]
        ugd = refs[14:22]
        uld = refs[22:30]
        sd = refs[30:38]
        ud = refs[38:46]
        pred_o, ul_o, cl_o = refs[46], refs[47], refs[48]
        _final_body(hc_r, gid_r, s2ss, s2su, fcr, ugd, uld, sd, ud,
                    pred_o, ul_o, cl_o)

    return pl.pallas_call(
        body,
        in_specs=[full(a) for a in args],
        out_specs=[
            pl.BlockSpec((G, 1), lambda: (0, 0)),
            pl.BlockSpec((1, 1), lambda: (0, 0)),
            pl.BlockSpec((1, 1), lambda: (0, 0)),
        ],
        out_shape=[
            jax.ShapeDtypeStruct((G, 1), _f32),
            jax.ShapeDtypeStruct((1, 1), _f32),
            jax.ShapeDtypeStruct((1, 1), _f32),
        ],
    )(*args)


def kernel(nfeat, efeat, edge_index, graph_id, params):
    ps, pu = params['sup'], params['unsup']
    src = edge_index[0].astype(jnp.int32)
    dst = edge_index[1].astype(jnp.int32)
    pad = E_PAD - E
    src2 = jnp.concatenate([src, jnp.zeros((pad,), jnp.int32)]).reshape(-1, CHUNK)
    dst2 = jnp.concatenate([dst, jnp.full((pad,), N, jnp.int32)]).reshape(-1, CHUNK)
    efeat_pad = jnp.concatenate([efeat, jnp.zeros((pad, 5), _f32)], axis=0)
    gid2d = graph_id.astype(jnp.int32).reshape(N, 1)

    degp = _sc_degree(dst2)
    hc = _tc_lin0(nfeat, ps, pu)

    for _ in range(3):
        out_src = _sc_gather(hc, src2)
        msg = _tc_matvec(out_src, efeat_pad, ps, pu)
        mp = _sc_scatter_add(msg, dst2, 2 * HID)
        hc = _tc_gru(mp[:, :N, :], degp[:, :N, :], hc, ps, pu)

    pred, ul, cl = _tc_final(hc, gid2d, params)
    return pred.reshape(-1), ul.reshape(()), cl.reshape(())


# speedup vs baseline: 1.8252x; 1.7515x over previous
import functools
import math

import jax
import jax.numpy as jnp
from jax import lax
from jax.experimental import pallas as pl
from jax.experimental.pallas import tpu as pltpu
from jax.experimental.pallas import tpu_sc as plsc

N = 10000
E = 160000
F_IN = 128
HID = 32
G = 64
LOG2 = math.log(2.0)

NW = 32
CHUNK = 128
C_PER_W = 40
E_PAD = NW * CHUNK * C_PER_W
N_SP = 10240
ROWS_PER_TILE = N_SP // 16

_f32 = jnp.float32


@functools.lru_cache(maxsize=None)
def _sc_gather_kernel(W):
    mesh = plsc.VectorSubcoreMesh(core_axis_name="c", subcore_axis_name="s")

    @functools.partial(
        pl.kernel, mesh=mesh,
        out_type=jax.ShapeDtypeStruct((E_PAD, W), _f32),
        compiler_params=pltpu.CompilerParams(use_tc_tiling_on_sc=False),
        scratch_types=[
            pltpu.VMEM((C_PER_W, CHUNK), jnp.int32),
            pltpu.VMEM((CHUNK, W), _f32),
            pltpu.SemaphoreType.DMA,
        ],
    )
    def k(table_hbm, idx_hbm, out_hbm, idxs_v, rows_v, sem):
        c = lax.axis_index("c")
        s = lax.axis_index("s")
        wid = s * 2 + c
        pltpu.sync_copy(idx_hbm.at[pl.ds(wid * C_PER_W, C_PER_W)], idxs_v)

        def body(j, carry):
            pltpu.async_copy(table_hbm.at[idxs_v.at[j]], rows_v, sem).wait()
            pltpu.sync_copy(rows_v, out_hbm.at[pl.ds((wid * C_PER_W + j) * CHUNK, CHUNK)])
            return carry

        lax.fori_loop(0, C_PER_W, body, 0)

    return k


def _sc_gather(table, idx2):
    return _sc_gather_kernel(table.shape[1])(table, idx2)


@functools.lru_cache(maxsize=None)
def _sc_scatter_kernel(W):
    mesh = plsc.VectorSubcoreMesh(core_axis_name="c", subcore_axis_name="s")

    @functools.partial(
        pl.kernel, mesh=mesh,
        out_type=jax.ShapeDtypeStruct((2, N_SP, W), _f32),
        compiler_params=pltpu.CompilerParams(use_tc_tiling_on_sc=False),
        scratch_types=[
            pltpu.VMEM((C_PER_W, CHUNK), jnp.int32),
            pltpu.VMEM((CHUNK, W), _f32),
            pltpu.VMEM_SHARED((N_SP, W), _f32),
        ],
    )
    def k(rows_hbm, idx_hbm, z_hbm, out_hbm, idxs_v, rows_v, acc_sh):
        c = lax.axis_index("c")
        s = lax.axis_index("s")
        wid = s * 2 + c
        pltpu.sync_copy(z_hbm, acc_sh.at[pl.ds(s * ROWS_PER_TILE, ROWS_PER_TILE)])
        pltpu.sync_copy(idx_hbm.at[pl.ds(wid * C_PER_W, C_PER_W)], idxs_v)
        plsc.subcore_barrier()

        def body(j, carry):
            pltpu.sync_copy(
                rows_hbm.at[pl.ds((wid * C_PER_W + j) * CHUNK, CHUNK)], rows_v)
            pltpu.sync_copy(rows_v, acc_sh.at[idxs_v.at[j]], add=True)
            return carry

        lax.fori_loop(0, C_PER_W, body, 0)
        plsc.subcore_barrier()
        pltpu.sync_copy(
            acc_sh.at[pl.ds(s * ROWS_PER_TILE, ROWS_PER_TILE)],
            out_hbm.at[c, pl.ds(s * ROWS_PER_TILE, ROWS_PER_TILE)])

    return k


def _sc_scatter_add(rows, idx2, W):
    zrows = jnp.zeros((ROWS_PER_TILE, W), _f32)
    return _sc_scatter_kernel(W)(rows, idx2, zrows)


def _sc_degree(idx2):
    mesh = plsc.VectorSubcoreMesh(core_axis_name="c", subcore_axis_name="s")
    Wd = 16
    zrows = jnp.zeros((ROWS_PER_TILE, Wd), _f32)
    ones = jnp.ones((CHUNK, Wd), _f32)

    @functools.partial(
        pl.kernel, mesh=mesh,
        out_type=jax.ShapeDtypeStruct((2, N_SP, Wd), _f32),
        compiler_params=pltpu.CompilerParams(use_tc_tiling_on_sc=False),
        scratch_types=[
            pltpu.VMEM((C_PER_W, CHUNK), jnp.int32),
            pltpu.VMEM((CHUNK, Wd), _f32),
            pltpu.VMEM_SHARED((N_SP, Wd), _f32),
        ],
    )
    def k(idx_hbm, z_hbm, ones_hbm, out_hbm, idxs_v, ones_v, acc_sh):
        c = lax.axis_index("c")
        s = lax.axis_index("s")
        wid = s * 2 + c
        pltpu.sync_copy(z_hbm, acc_sh.at[pl.ds(s * ROWS_PER_TILE, ROWS_PER_TILE)])
        pltpu.sync_copy(idx_hbm.at[pl.ds(wid * C_PER_W, C_PER_W)], idxs_v)
        pltpu.sync_copy(ones_hbm, ones_v)
        plsc.subcore_barrier()

        def body(j, carry):
            pltpu.sync_copy(ones_v, acc_sh.at[idxs_v.at[j]], add=True)
            return carry

        lax.fori_loop(0, C_PER_W, body, 0)
        plsc.subcore_barrier()
        pltpu.sync_copy(
            acc_sh.at[pl.ds(s * ROWS_PER_TILE, ROWS_PER_TILE)],
            out_hbm.at[c, pl.ds(s * ROWS_PER_TILE, ROWS_PER_TILE)])

    return k(idx2, zrows, ones)


BN = 1000
BE = 640


def _relu(x):
    return jnp.maximum(x, 0.0)


def _lin0_body(nf, ws, bs, wu, bu, o):
    x = nf[...]
    a = _relu(jnp.dot(x, ws[...], preferred_element_type=_f32) + bs[...])
    b = _relu(jnp.dot(x, wu[...], preferred_element_type=_f32) + bu[...])
    o[...] = jnp.concatenate([a, b], axis=1)


def _tc_lin0(nfeat, ps, pu):
    return pl.pallas_call(
        _lin0_body,
        grid=(N // BN,),
        in_specs=[
            pl.BlockSpec((BN, F_IN), lambda i: (i, 0)),
            pl.BlockSpec((F_IN, HID), lambda i: (0, 0)),
            pl.BlockSpec((1, HID), lambda i: (0, 0)),
            pl.BlockSpec((F_IN, HID), lambda i: (0, 0)),
            pl.BlockSpec((1, HID), lambda i: (0, 0)),
        ],
        out_specs=pl.BlockSpec((BN, 2 * HID), lambda i: (i, 0)),
        out_shape=jax.ShapeDtypeStruct((N, 2 * HID), _f32),
    )(nfeat, ps['lin0_w'], ps['lin0_b'].reshape(1, -1),
      pu['lin0_w'], pu['lin0_b'].reshape(1, -1))


def _edgenet_body(ef, w1s, b1s, w2s, b2s, w1u, b1u, w2u, b2u, os_, ou_):
    x = ef[...]
    hs = _relu(jnp.dot(x, w1s[...], preferred_element_type=_f32) + b1s[...])
    os_[...] = jnp.dot(hs, w2s[...], preferred_element_type=_f32) + b2s[...]
    hu = _relu(jnp.dot(x, w1u[...], preferred_element_type=_f32) + b1u[...])
    ou_[...] = jnp.dot(hu, w2u[...], preferred_element_type=_f32) + b2u[...]


def _tc_edgenet(efeat_pad, ps, pu):
    HH = HID * HID
    return pl.pallas_call(
        _edgenet_body,
        grid=(E_PAD // BE,),
        in_specs=[
            pl.BlockSpec((BE, 5), lambda i: (i, 0)),
            pl.BlockSpec((5, 128), lambda i: (0, 0)),
            pl.BlockSpec((1, 128), lambda i: (0, 0)),
            pl.BlockSpec((128, HH), lambda i: (0, 0)),
            pl.BlockSpec((1, HH), lambda i: (0, 0)),
            pl.BlockSpec((5, 128), lambda i: (0, 0)),
            pl.BlockSpec((1, 128), lambda i: (0, 0)),
            pl.BlockSpec((128, HH), lambda i: (0, 0)),
            pl.BlockSpec((1, HH), lambda i: (0, 0)),
        ],
        out_specs=[
            pl.BlockSpec((BE, HH), lambda i: (i, 0)),
            pl.BlockSpec((BE, HH), lambda i: (i, 0)),
        ],
        out_shape=[
            jax.ShapeDtypeStruct((E_PAD, HH), _f32),
            jax.ShapeDtypeStruct((E_PAD, HH), _f32),
        ],
    )(efeat_pad, ps['nn_w1'], ps['nn_b1'].reshape(1, -1),
      ps['nn_w2'], ps['nn_b2'].reshape(1, -1),
      pu['nn_w1'], pu['nn_b1'].reshape(1, -1),
      pu['nn_w2'], pu['nn_b2'].reshape(1, -1))


def _fold_sum(y):
    w = HID * HID
    while w > HID:
        w //= 2
        y = y[:, :w] + y[:, w:]
    return y


def _matvec_body(osrc, ef, rexp, w1s, b1s, w2s, b2s, w1u, b1u, w2u, b2u, o):
    x = osrc[...]
    e = ef[...]
    r = rexp[...]
    hs = _relu(jnp.dot(e, w1s[...], preferred_element_type=_f32) + b1s[...])
    ews = jnp.dot(hs, w2s[...], preferred_element_type=_f32) + b2s[...]
    hu = _relu(jnp.dot(e, w1u[...], preferred_element_type=_f32) + b1u[...])
    ewu = jnp.dot(hu, w2u[...], preferred_element_type=_f32) + b2u[...]
    xs = jnp.dot(x[:, :HID], r, precision=jax.lax.Precision.HIGHEST,
                 preferred_element_type=_f32)
    xu = jnp.dot(x[:, HID:], r, precision=jax.lax.Precision.HIGHEST,
                 preferred_element_type=_f32)
    accs = _fold_sum(xs * ews)
    accu = _fold_sum(xu * ewu)
    o[...] = jnp.concatenate([accs, accu], axis=1)


def _tc_matvec(out_src, efeat_pad, rexp, ps, pu):
    HH = HID * HID
    return pl.pallas_call(
        _matvec_body,
        grid=(E_PAD // BE,),
        in_specs=[
            pl.BlockSpec((BE, 2 * HID), lambda i: (i, 0)),
            pl.BlockSpec((BE, 5), lambda i: (i, 0)),
            pl.BlockSpec((HID, HH), lambda i: (0, 0)),
            pl.BlockSpec((5, 128), lambda i: (0, 0)),
            pl.BlockSpec((1, 128), lambda i: (0, 0)),
            pl.BlockSpec((128, HH), lambda i: (0, 0)),
            pl.BlockSpec((1, HH), lambda i: (0, 0)),
            pl.BlockSpec((5, 128), lambda i: (0, 0)),
            pl.BlockSpec((1, 128), lambda i: (0, 0)),
            pl.BlockSpec((128, HH), lambda i: (0, 0)),
            pl.BlockSpec((1, HH), lambda i: (0, 0)),
        ],
        out_specs=pl.BlockSpec((BE, 2 * HID), lambda i: (i, 0)),
        out_shape=jax.ShapeDtypeStruct((E_PAD, 2 * HID), _f32),
    )(out_src, efeat_pad, rexp,
      ps['nn_w1'], ps['nn_b1'].reshape(1, -1),
      ps['nn_w2'], ps['nn_b2'].reshape(1, -1),
      pu['nn_w1'], pu['nn_b1'].reshape(1, -1),
      pu['nn_w2'], pu['nn_b2'].reshape(1, -1))


def _gru_half(m, h, wi, bi, wh, bh):
    gi = jnp.dot(m, wi, preferred_element_type=_f32) + bi
    gh = jnp.dot(h, wh, preferred_element_type=_f32) + bh
    r = jax.nn.sigmoid(gi[:, :HID] + gh[:, :HID])
    z = jax.nn.sigmoid(gi[:, HID:2 * HID] + gh[:, HID:2 * HID])
    n = jnp.tanh(gi[:, 2 * HID:] + r * gh[:, 2 * HID:])
    return (1.0 - z) * n + z * h


def _gru_body(mp, dp, hc, cbs, cbu, wis, bis, whs, bhs, wiu, biu, whu, bhu, o):
    deg = dp[0][:, 0:1] + dp[1][:, 0:1]
    rdeg = 1.0 / jnp.maximum(deg, 1.0)
    msum = (mp[0] + mp[1]) * rdeg
    m_s = _relu(msum[:, :HID] + cbs[...])
    m_u = _relu(msum[:, HID:] + cbu[...])
    h_s = hc[...][:, :HID]
    h_u = hc[...][:, HID:]
    ns = _gru_half(m_s, h_s, wis[...], bis[...], whs[...], bhs[...])
    nu = _gru_half(m_u, h_u, wiu[...], biu[...], whu[...], bhu[...])
    o[...] = jnp.concatenate([ns, nu], axis=1)


def _tc_gru(mp, degp, hc, ps, pu):
    return pl.pallas_call(
        _gru_body,
        grid=(N // BN,),
        in_specs=[
            pl.BlockSpec((2, BN, 2 * HID), lambda i: (0, i, 0)),
            pl.BlockSpec((2, BN, 16), lambda i: (0, i, 0)),
            pl.BlockSpec((BN, 2 * HID), lambda i: (i, 0)),
            pl.BlockSpec((1, HID), lambda i: (0, 0)),
            pl.BlockSpec((1, HID), lambda i: (0, 0)),
            pl.BlockSpec((HID, 3 * HID), lambda i: (0, 0)),
            pl.BlockSpec((1, 3 * HID), lambda i: (0, 0)),
            pl.BlockSpec((HID, 3 * HID), lambda i: (0, 0)),
            pl.BlockSpec((1, 3 * HID), lambda i: (0, 0)),
            pl.BlockSpec((HID, 3 * HID), lambda i: (0, 0)),
            pl.BlockSpec((1, 3 * HID), lambda i: (0, 0)),
            pl.BlockSpec((HID, 3 * HID), lambda i: (0, 0)),
            pl.BlockSpec((1, 3 * HID), lambda i: (0, 0)),
        ],
        out_specs=pl.BlockSpec((BN, 2 * HID), lambda i: (i, 0)),
        out_shape=jax.ShapeDtypeStruct((N, 2 * HID), _f32),
    )(mp, degp, hc,
      ps['conv_b'].reshape(1, -1), pu['conv_b'].reshape(1, -1),
      ps['gru_wi'], ps['gru_bi'].reshape(1, -1),
      ps['gru_wh'], ps['gru_bh'].reshape(1, -1),
      pu['gru_wi'], pu['gru_bi'].reshape(1, -1),
      pu['gru_wh'], pu['gru_bh'].reshape(1, -1))


def _softplus(x):
    return jnp.maximum(x, 0.0) + jnp.log(1.0 + jnp.exp(-jnp.abs(x)))


def _pos_exp(x):
    return LOG2 - _softplus(-x)


def _neg_exp(x):
    return _softplus(-x) + x - LOG2


def _ffnn_in(x, w1, b1, w2, b2, w3, b3, jw, jb):
    h = _relu(jnp.dot(x, w1, preferred_element_type=_f32) + b1)
    h = _relu(jnp.dot(h, w2, preferred_element_type=_f32) + b2)
    h = _relu(jnp.dot(h, w3, preferred_element_type=_f32) + b3)
    return h + jnp.dot(x, jw, preferred_element_type=_f32) + jb


def _set2set(out, oh, wi, bi, wh, bh):
    q_star = jnp.zeros((G, 2 * HID), _f32)
    hh = jnp.zeros((G, HID), _f32)
    cc = jnp.zeros((G, HID), _f32)
    for _ in range(3):
        gates = (jnp.dot(q_star, wi, preferred_element_type=_f32) + bi
                 + jnp.dot(hh, wh, preferred_element_type=_f32) + bh)
        i_, f_, g_, o_ = (gates[:, :HID], gates[:, HID:2 * HID],
                          gates[:, 2 * HID:3 * HID], gates[:, 3 * HID:])
        cc = jax.nn.sigmoid(f_) * cc + jax.nn.sigmoid(i_) * jnp.tanh(g_)
        hh = jax.nn.sigmoid(o_) * jnp.tanh(cc)
        qn = jnp.dot(oh, hh, preferred_element_type=_f32)
        e = jnp.sum(out * qn, axis=1, keepdims=True)
        em = jnp.max(jnp.where(oh > 0.0, e, -1e30), axis=0, keepdims=True)
        ee = jnp.exp(e - jnp.dot(oh, em.T, preferred_element_type=_f32))
        denom = lax.dot_general(oh, ee, (((0,), (0,)), ((), ())),
                                preferred_element_type=_f32)
        inv = 1.0 / jnp.maximum(denom, 1e-30)
        a = ee * jnp.dot(oh, inv, preferred_element_type=_f32)
        r = lax.dot_general(oh, a * out, (((0,), (0,)), ((), ())),
                            preferred_element_type=_f32)
        q_star = jnp.concatenate([hh, r], axis=1)
    return q_star


def _final_body(hc, gid, s2s_s, s2s_u, fc, f_ugd, f_uld, f_sd, f_ud,
                pred_o, ul_o, cl_o):
    out_s = hc[...][:, :HID]
    out_u = hc[...][:, HID:]
    g = gid[...]
    oh = (lax.broadcasted_iota(jnp.int32, (N, G), 1) == g).astype(_f32)

    sup_emb = _set2set(out_s, oh, *[r[...] for r in s2s_s])
    unsup_emb = _set2set(out_u, oh, *[r[...] for r in s2s_u])

    fc1_w, fc1_b, fc2_w, fc2_b = [r[...] for r in fc]
    pred = (jnp.dot(_relu(jnp.dot(sup_emb, fc1_w,
                                  preferred_element_type=_f32) + fc1_b),
                    fc2_w, preferred_element_type=_f32) + fc2_b)
    pred_o[...] = pred

    g_enc = _ffnn_in(unsup_emb, *[r[...] for r in f_ugd])
    l_enc = _ffnn_in(out_u, *[r[...] for r in f_uld])
    sup_g = _ffnn_in(sup_emb, *[r[...] for r in f_sd])
    unsup_g = _ffnn_in(unsup_emb, *[r[...] for r in f_ud])

    res = lax.dot_general(l_enc, g_enc, (((1,), (1,)), ((), ())),
                          preferred_element_type=_f32)
    e_pos = jnp.sum(_pos_exp(res * oh)) / N
    e_neg = jnp.sum(_neg_exp(res * (1.0 - oh))) / (N * (G - 1))
    ul_o[...] = jnp.broadcast_to(e_neg - e_pos, (1, 1))

    res2 = lax.dot_general(sup_g, unsup_g, (((1,), (1,)), ((), ())),
                           preferred_element_type=_f32)
    eye = (lax.broadcasted_iota(jnp.int32, (G, G), 0)
           == lax.broadcasted_iota(jnp.int32, (G, G), 1)).astype(_f32)
    e_pos2 = jnp.sum(_pos_exp(res2 * eye)) / G
    e_neg2 = jnp.sum(_neg_exp(res2 * (1.0 - eye))) / (G * (G - 1))
    cl_o[...] = jnp.broadcast_to(e_neg2 - e_pos2, (1, 1))


def _tc_final(hc, gid2d, params):
    ps, pu = params['sup'], params['unsup']
    s2s_s = [ps['s2s_wi'], ps['s2s_bi'].reshape(1, -1),
             ps['s2s_wh'], ps['s2s_bh'].reshape(1, -1)]
    s2s_u = [pu['s2s_wi'], pu['s2s_bi'].reshape(1, -1),
             pu['s2s_wh'], pu['s2s_bh'].reshape(1, -1)]
    fc = [params['fc1_w'], params['fc1_b'].reshape(1, -1),
          params['fc2_w'], params['fc2_b'].reshape(1, -1)]

    def ffnn_list(p):
        return [p['w1'], p['b1'].reshape(1, -1), p['w2'], p['b2'].reshape(1, -1),
                p['w3'], p['b3'].reshape(1, -1), p['jw'], p['jb'].reshape(1, -1)]

    full = lambda x: pl.BlockSpec(x.shape, lambda: tuple(0 for _ in x.shape))
    args = [hc, gid2d] + s2s_s + s2s_u + fc + \
        ffnn_list(params['ugd']) + ffnn_list(params['uld']) + \
        ffnn_list(params['sd']) + ffnn_list(params['ud'])

    def body(*refs):
        hc_r, gid_r = refs[0], refs[1]
        s2ss = refs[2:6]
        s2su = refs[6:10]
        fcr = refs[10:14]
        ugd = refs[14:22]
        uld = refs[22:30]
        sd = refs[30:38]
        ud = refs[38:46]
        pred_o, ul_o, cl_o = refs[46], refs[47], refs[48]
        _final_body(hc_r, gid_r, s2ss, s2su, fcr, ugd, uld, sd, ud,
                    pred_o, ul_o, cl_o)

    return pl.pallas_call(
        body,
        in_specs=[full(a) for a in args],
        out_specs=[
            pl.BlockSpec((G, 1), lambda: (0, 0)),
            pl.BlockSpec((1, 1), lambda: (0, 0)),
            pl.BlockSpec((1, 1), lambda: (0, 0)),
        ],
        out_shape=[
            jax.ShapeDtypeStruct((G, 1), _f32),
            jax.ShapeDtypeStruct((1, 1), _f32),
            jax.ShapeDtypeStruct((1, 1), _f32),
        ],
    )(*args)


def kernel(nfeat, efeat, edge_index, graph_id, params):
    ps, pu = params['sup'], params['unsup']
    src = edge_index[0].astype(jnp.int32)
    dst = edge_index[1].astype(jnp.int32)
    pad = E_PAD - E
    src2 = jnp.concatenate([src, jnp.zeros((pad,), jnp.int32)]).reshape(-1, CHUNK)
    dst2 = jnp.concatenate([dst, jnp.full((pad,), N, jnp.int32)]).reshape(-1, CHUNK)
    efeat_pad = jnp.concatenate([efeat, jnp.zeros((pad, 5), _f32)], axis=0)
    gid2d = graph_id.astype(jnp.int32).reshape(N, 1)
    rexp = jnp.repeat(jnp.eye(HID, dtype=_f32), HID, axis=1)

    degp = _sc_degree(dst2)
    hc = _tc_lin0(nfeat, ps, pu)

    for _ in range(3):
        out_src = _sc_gather(hc, src2)
        msg = _tc_matvec(out_src, efeat_pad, rexp, ps, pu)
        mp = _sc_scatter_add(msg, dst2, 2 * HID)
        hc = _tc_gru(mp[:, :N, :], degp[:, :N, :], hc, ps, pu)

    pred, ul, cl = _tc_final(hc, gid2d, params)
    return pred.reshape(-1), ul.reshape(()), cl.reshape(())
